# baseline (speedup 1.0000x reference)
import functools

import jax
import jax.numpy as jnp
from jax import lax
from jax.experimental import pallas as pl
from jax.experimental.pallas import tpu as pltpu
from jax.experimental.pallas import tpu_sc as plsc

_N = 10000
_E = 320000
_D = 128
_NC = 2
_NS = 16
_K = 128
_EP = 327680
_NCHUNK = _EP // _K
_CPT = _NCHUNK // (_NC * _NS)
_NA = 10016
_RPS = 1000
_RB = 1000

_MESH = dict(core_axis_name="c", subcore_axis_name="s", num_cores=_NC,
             num_subcores=_NS)


def _sc_degrees(sd):
    per_tile = _EP // _NS
    n_chunks = per_tile // _K

    @functools.partial(
        pl.kernel,
        out_type=jax.ShapeDtypeStruct((2 * _N,), jnp.float32),
        mesh=plsc.VectorSubcoreMesh(**_MESH),
        scratch_types=[
            pltpu.VMEM((_K,), jnp.int32),
            pltpu.VMEM((_K,), jnp.float32),
            pltpu.VMEM((_RPS,), jnp.float32),
            pltpu.VMEM_SHARED((_NA,), jnp.float32),
        ],
    )
    def deg_kernel(sd_hbm, out_hbm, idx_v, ones_v, zero_v, acc_sh):
        ci = lax.axis_index("c")
        si = lax.axis_index("s")

        @pl.loop(0, _K, step=16)
        def _(i):
            ones_v[pl.ds(i, 16)] = jnp.full((16,), 1.0, jnp.float32)

        @pl.when(si < 10)
        def _():
            @pl.loop(0, _RPS, step=16)
            def _(i):
                zero_v[pl.ds(i, 16)] = jnp.full((16,), 0.0, jnp.float32)

            pltpu.sync_copy(zero_v, acc_sh.at[pl.ds(si * _RPS, _RPS)])

        plsc.subcore_barrier()
        base = ci * _EP + si * per_tile

        @pl.loop(0, n_chunks)
        def _(c):
            pltpu.sync_copy(sd_hbm.at[pl.ds(base + c * _K, _K)], idx_v)
            pltpu.sync_copy(ones_v, acc_sh.at[idx_v], add=True)

        plsc.subcore_barrier()

        @pl.when(si < 10)
        def _():
            pltpu.sync_copy(acc_sh.at[pl.ds(si * _RPS, _RPS)], zero_v)
            pltpu.sync_copy(zero_v,
                            out_hbm.at[pl.ds(ci * _N + si * _RPS, _RPS)])

    return deg_kernel(sd)


def _sc_agg(g, srcp, dstp, zeros_blk):
    per_tile = _EP // (_NC * _NS)

    @functools.partial(
        pl.kernel,
        out_type=jax.ShapeDtypeStruct((2 * _N, _D), jnp.float32),
        mesh=plsc.VectorSubcoreMesh(**_MESH),
        scratch_types=[
            pltpu.VMEM((_K,), jnp.int32),
            pltpu.VMEM((_K,), jnp.int32),
            pltpu.VMEM((_K, _D), jnp.float32),
            pltpu.VMEM_SHARED((_NA, _D), jnp.float32),
        ],
    )
    def agg_kernel(g_hbm, src_hbm, dst_hbm, zeros_hbm, out_hbm, sidx_v,
                   didx_v, rows_v, acc_sh):
        ci = lax.axis_index("c")
        si = lax.axis_index("s")

        @pl.when(si < 10)
        def _():
            pltpu.sync_copy(zeros_hbm, acc_sh.at[pl.ds(si * _RPS, _RPS)])

        plsc.subcore_barrier()
        base = (ci * _NS + si) * per_tile

        @pl.loop(0, _CPT)
        def _(q):
            off = base + q * _K
            pltpu.sync_copy(src_hbm.at[pl.ds(off, _K)], sidx_v)
            pltpu.sync_copy(dst_hbm.at[pl.ds(off, _K)], didx_v)
            pltpu.sync_copy(g_hbm.at[sidx_v], rows_v)
            pltpu.sync_copy(rows_v, acc_sh.at[didx_v], add=True)

        plsc.subcore_barrier()

        @pl.when(si < 10)
        def _():
            pltpu.sync_copy(acc_sh.at[pl.ds(si * _RPS, _RPS)],
                            out_hbm.at[pl.ds(ci * _N + si * _RPS, _RPS)])

    return agg_kernel(g, srcp, dstp, zeros_blk)


def _norm(deg):
    return jnp.where(deg > 0, lax.rsqrt(jnp.maximum(deg, 1.0)), 0.0)


def _mm(a, b):
    return lax.dot_general(a, b, (((1,), (0,)), ((), ())),
                           precision=lax.Precision.HIGHEST,
                           preferred_element_type=jnp.float32)


def _tc_mm_scale(x, W, degout):
    def body(x_ref, w_ref, d_ref, o_ref):
        o_ref[...] = _mm(x_ref[...], w_ref[...]) * _norm(d_ref[...])

    return pl.pallas_call(
        body,
        grid=(_N // _RB,),
        in_specs=[pl.BlockSpec((_RB, _D), lambda i: (i, 0)),
                  pl.BlockSpec((_D, _D), lambda i: (0, 0)),
                  pl.BlockSpec((_RB, 1), lambda i: (i, 0))],
        out_specs=pl.BlockSpec((_RB, _D), lambda i: (i, 0)),
        out_shape=jax.ShapeDtypeStruct((_N, _D), jnp.float32),
    )(x, W, degout)


def _tc_mid(agg, degin, b1, W2, degout):
    def body(a_ref, di_ref, b_ref, w_ref, do_ref, o_ref):
        s = a_ref[0] + a_ref[1]
        h = jnp.maximum(s * _norm(di_ref[...]) + b_ref[...], 0.0)
        o_ref[...] = _mm(h, w_ref[...]) * _norm(do_ref[...])

    return pl.pallas_call(
        body,
        grid=(_N // _RB,),
        in_specs=[pl.BlockSpec((_NC, _RB, _D), lambda i: (0, i, 0)),
                  pl.BlockSpec((_RB, 1), lambda i: (i, 0)),
                  pl.BlockSpec((1, _D), lambda i: (0, 0)),
                  pl.BlockSpec((_D, _D), lambda i: (0, 0)),
                  pl.BlockSpec((_RB, 1), lambda i: (i, 0))],
        out_specs=pl.BlockSpec((_RB, _D), lambda i: (i, 0)),
        out_shape=jax.ShapeDtypeStruct((_N, _D), jnp.float32),
    )(agg, degin, b1, W2, degout)


def _tc_fin(agg, degin, b2):
    def body(a_ref, di_ref, b_ref, o_ref):
        o_ref[...] = (a_ref[0] + a_ref[1]) * _norm(di_ref[...]) + b_ref[...]

    return pl.pallas_call(
        body,
        grid=(_N // _RB,),
        in_specs=[pl.BlockSpec((_NC, _RB, _D), lambda i: (0, i, 0)),
                  pl.BlockSpec((_RB, 1), lambda i: (i, 0)),
                  pl.BlockSpec((1, _D), lambda i: (0, 0))],
        out_specs=pl.BlockSpec((_RB, _D), lambda i: (i, 0)),
        out_shape=jax.ShapeDtypeStruct((_N, _D), jnp.float32),
    )(agg, degin, b2)


def kernel(x, edge_index, W1, b1, W2, b2):
    src = edge_index[0]
    dst = edge_index[1]
    pad = _EP - _E
    padz = jnp.zeros((pad,), jnp.int32)
    padn = jnp.full((pad,), _N, jnp.int32)
    srcp = jnp.concatenate([src, padz])
    dstp = jnp.concatenate([dst, padn])
    sd = jnp.concatenate([src, padn, dst, padn])
    zeros_blk = jnp.zeros((_RPS, _D), jnp.float32)

    deg = _sc_degrees(sd).reshape(_NC, _N, 1)
    degout = deg[0]
    degin = deg[1]

    g1 = _tc_mm_scale(x, W1, degout)
    agg1 = _sc_agg(g1, srcp, dstp, zeros_blk).reshape(_NC, _N, _D)
    g2 = _tc_mid(agg1, degin, b1.reshape(1, _D), W2, degout)
    agg2 = _sc_agg(g2, srcp, dstp, zeros_blk).reshape(_NC, _N, _D)
    return _tc_fin(agg2, degin, b2.reshape(1, _D))


# speedup vs baseline: 2.4506x; 2.4506x over previous
import functools

import jax
import jax.numpy as jnp
from jax import lax
from jax.experimental import pallas as pl
from jax.experimental.pallas import tpu as pltpu
from jax.experimental.pallas import tpu_sc as plsc

_N = 10000
_E = 320000
_D = 128
_NC = 2
_NS = 16
_K = 80
_CPT = _E // (_NC * _NS * _K)
_RPS = 1000
_RB = 1000

_MESH = dict(core_axis_name="c", subcore_axis_name="s", num_cores=_NC,
             num_subcores=_NS)


def _sc_degrees(sd):
    per_tile = _E // _NS
    n_chunks = per_tile // _K
    n_grp = n_chunks // 2

    @functools.partial(
        pl.kernel,
        out_type=jax.ShapeDtypeStruct((2 * _N,), jnp.float32),
        mesh=plsc.VectorSubcoreMesh(**_MESH),
        scratch_types=[
            pltpu.VMEM((2, _K), jnp.int32),
            pltpu.VMEM((_K,), jnp.float32),
            pltpu.VMEM((_RPS,), jnp.float32),
            pltpu.VMEM_SHARED((_N,), jnp.float32),
            pltpu.SemaphoreType.DMA,
            pltpu.SemaphoreType.DMA,
        ],
    )
    def deg_kernel(sd_hbm, out_hbm, idx_v, ones_v, zero_v, acc_sh, si0, si1):
        ci = lax.axis_index("c")
        si = lax.axis_index("s")
        sem = (si0, si1)

        @pl.loop(0, _K, step=16)
        def _(i):
            ones_v[pl.ds(i, 16)] = jnp.full((16,), 1.0, jnp.float32)

        @pl.when(si < 10)
        def _():
            @pl.loop(0, _RPS, step=16)
            def _(i):
                zero_v[pl.ds(i, 16)] = jnp.full((16,), 0.0, jnp.float32)

            pltpu.sync_copy(zero_v, acc_sh.at[pl.ds(si * _RPS, _RPS)])

        plsc.subcore_barrier()
        base = ci * _E + si * per_tile

        def idx_src(q):
            return sd_hbm.at[pl.ds(base + q * _K, _K)]

        pltpu.sync_copy(idx_src(0), idx_v.at[0])

        @pl.loop(0, n_grp)
        def _(grp):
            for b in (0, 1):
                nb = 1 - b
                q = 2 * grp + b
                if b == 0:
                    pltpu.async_copy(idx_src(q + 1), idx_v.at[nb], sem[nb])
                else:
                    @pl.when(grp < n_grp - 1)
                    def _():
                        pltpu.async_copy(idx_src(q + 1), idx_v.at[nb],
                                         sem[nb])

                if b == 0:
                    @pl.when(grp > 0)
                    def _():
                        pltpu.make_async_copy(idx_src(q), idx_v.at[b],
                                              sem[b]).wait()
                else:
                    pltpu.make_async_copy(idx_src(q), idx_v.at[b],
                                          sem[b]).wait()

                pltpu.sync_copy(ones_v, acc_sh.at[idx_v.at[b]], add=True)

        plsc.subcore_barrier()

        @pl.when(si < 10)
        def _():
            pltpu.sync_copy(acc_sh.at[pl.ds(si * _RPS, _RPS)], zero_v)
            pltpu.sync_copy(zero_v,
                            out_hbm.at[pl.ds(ci * _N + si * _RPS, _RPS)])

    return deg_kernel(sd)


def _sc_agg(g, src, dst, zeros_blk):
    per_tile = _E // (_NC * _NS)
    n_grp = (_CPT - 1) // 2

    @functools.partial(
        pl.kernel,
        out_type=jax.ShapeDtypeStruct((2 * _N, _D), jnp.float32),
        mesh=plsc.VectorSubcoreMesh(**_MESH),
        scratch_types=[
            pltpu.VMEM((2, _K), jnp.int32),
            pltpu.VMEM((2, _K), jnp.int32),
            pltpu.VMEM((2, _K, _D), jnp.float32),
            pltpu.VMEM_SHARED((_N, _D), jnp.float32),
            pltpu.SemaphoreType.DMA,
            pltpu.SemaphoreType.DMA,
            pltpu.SemaphoreType.DMA,
            pltpu.SemaphoreType.DMA,
        ],
    )
    def agg_kernel(g_hbm, src_hbm, dst_hbm, zeros_hbm, out_hbm, sidx_v,
                   didx_v, rows_v, acc_sh, sg0, sg1, ss0, ss1):
        ci = lax.axis_index("c")
        si = lax.axis_index("s")
        sg = (sg0, sg1)
        ss = (ss0, ss1)

        @pl.when(si < 10)
        def _():
            pltpu.sync_copy(zeros_hbm, acc_sh.at[pl.ds(si * _RPS, _RPS)])

        plsc.subcore_barrier()
        base = (ci * _NS + si) * per_tile

        def fetch(q, buf):
            off = base + q * _K
            pltpu.sync_copy(src_hbm.at[pl.ds(off, _K)], sidx_v.at[buf])
            pltpu.sync_copy(dst_hbm.at[pl.ds(off, _K)], didx_v.at[buf])
            pltpu.async_copy(g_hbm.at[sidx_v.at[buf]], rows_v.at[buf],
                             sg[buf])

        def wait_gather(buf):
            pltpu.make_async_copy(g_hbm.at[sidx_v.at[buf]], rows_v.at[buf],
                                  sg[buf]).wait()

        def scatter(buf):
            pltpu.async_copy(rows_v.at[buf], acc_sh.at[didx_v.at[buf]],
                             ss[buf], add=True)

        def wait_scatter(buf):
            pltpu.make_async_copy(rows_v.at[buf], acc_sh.at[didx_v.at[buf]],
                                  ss[buf]).wait()

        fetch(0, 0)

        @pl.loop(0, n_grp)
        def _(grp):
            for b in (0, 1):
                nb = 1 - b
                q = 2 * grp + b
                if b == 0:
                    @pl.when(grp > 0)
                    def _():
                        wait_scatter(nb)
                else:
                    wait_scatter(nb)

                fetch(q + 1, nb)

                wait_gather(b)
                scatter(b)

        wait_scatter(1)
        wait_gather(0)
        scatter(0)
        wait_scatter(0)
        plsc.subcore_barrier()

        @pl.when(si < 10)
        def _():
            pltpu.sync_copy(acc_sh.at[pl.ds(si * _RPS, _RPS)],
                            out_hbm.at[pl.ds(ci * _N + si * _RPS, _RPS)])

    return agg_kernel(g, src, dst, zeros_blk)


def _norm(deg):
    return jnp.where(deg > 0, lax.rsqrt(jnp.maximum(deg, 1.0)), 0.0)


def _mm(a, b):
    return lax.dot_general(a, b, (((1,), (0,)), ((), ())),
                           precision=lax.Precision.HIGHEST,
                           preferred_element_type=jnp.float32)


def _tc_mm_scale(x, W, degout):
    def body(x_ref, w_ref, d_ref, o_ref):
        o_ref[...] = _mm(x_ref[...], w_ref[...]) * _norm(d_ref[...])

    return pl.pallas_call(
        body,
        grid=(_N // _RB,),
        in_specs=[pl.BlockSpec((_RB, _D), lambda i: (i, 0)),
                  pl.BlockSpec((_D, _D), lambda i: (0, 0)),
                  pl.BlockSpec((_RB, 1), lambda i: (i, 0))],
        out_specs=pl.BlockSpec((_RB, _D), lambda i: (i, 0)),
        out_shape=jax.ShapeDtypeStruct((_N, _D), jnp.float32),
    )(x, W, degout)


def _tc_mid(agg, degin, b1, W2, degout):
    def body(a_ref, di_ref, b_ref, w_ref, do_ref, o_ref):
        s = a_ref[0] + a_ref[1]
        h = jnp.maximum(s * _norm(di_ref[...]) + b_ref[...], 0.0)
        o_ref[...] = _mm(h, w_ref[...]) * _norm(do_ref[...])

    return pl.pallas_call(
        body,
        grid=(_N // _RB,),
        in_specs=[pl.BlockSpec((_NC, _RB, _D), lambda i: (0, i, 0)),
                  pl.BlockSpec((_RB, 1), lambda i: (i, 0)),
                  pl.BlockSpec((1, _D), lambda i: (0, 0)),
                  pl.BlockSpec((_D, _D), lambda i: (0, 0)),
                  pl.BlockSpec((_RB, 1), lambda i: (i, 0))],
        out_specs=pl.BlockSpec((_RB, _D), lambda i: (i, 0)),
        out_shape=jax.ShapeDtypeStruct((_N, _D), jnp.float32),
    )(agg, degin, b1, W2, degout)


def _tc_fin(agg, degin, b2):
    def body(a_ref, di_ref, b_ref, o_ref):
        o_ref[...] = (a_ref[0] + a_ref[1]) * _norm(di_ref[...]) + b_ref[...]

    return pl.pallas_call(
        body,
        grid=(_N // _RB,),
        in_specs=[pl.BlockSpec((_NC, _RB, _D), lambda i: (0, i, 0)),
                  pl.BlockSpec((_RB, 1), lambda i: (i, 0)),
                  pl.BlockSpec((1, _D), lambda i: (0, 0))],
        out_specs=pl.BlockSpec((_RB, _D), lambda i: (i, 0)),
        out_shape=jax.ShapeDtypeStruct((_N, _D), jnp.float32),
    )(agg, degin, b2)


def kernel(x, edge_index, W1, b1, W2, b2):
    src = edge_index[0]
    dst = edge_index[1]
    sd = jnp.concatenate([src, dst])
    zeros_blk = jnp.zeros((_RPS, _D), jnp.float32)

    deg = _sc_degrees(sd).reshape(_NC, _N, 1)
    degout = deg[0]
    degin = deg[1]

    g1 = _tc_mm_scale(x, W1, degout)
    agg1 = _sc_agg(g1, src, dst, zeros_blk).reshape(_NC, _N, _D)
    g2 = _tc_mid(agg1, degin, b1.reshape(1, _D), W2, degout)
    agg2 = _sc_agg(g2, src, dst, zeros_blk).reshape(_NC, _N, _D)
    return _tc_fin(agg2, degin, b2.reshape(1, _D))


# speedup vs baseline: 2.7362x; 1.1165x over previous
import functools

import jax
import jax.numpy as jnp
from jax import lax
from jax.experimental import pallas as pl
from jax.experimental.pallas import tpu as pltpu
from jax.experimental.pallas import tpu_sc as plsc

_N = 10000
_E = 320000
_D = 128
_NC = 2
_NS = 16
_K = 80
_CPT = _E // (_NC * _NS * _K)
_RPS = 1000
_RB = 1000

_MESH = dict(core_axis_name="c", subcore_axis_name="s", num_cores=_NC,
             num_subcores=_NS)


def _sc_degrees(sd):
    per_tile = _E // _NS
    n_chunks = per_tile // _K
    n_grp = n_chunks // 2

    @functools.partial(
        pl.kernel,
        out_type=jax.ShapeDtypeStruct((2 * _N,), jnp.float32),
        mesh=plsc.VectorSubcoreMesh(**_MESH),
        scratch_types=[
            pltpu.VMEM((2, _K), jnp.int32),
            pltpu.VMEM((_K,), jnp.float32),
            pltpu.VMEM((_RPS,), jnp.float32),
            pltpu.VMEM_SHARED((_N,), jnp.float32),
            pltpu.SemaphoreType.DMA,
            pltpu.SemaphoreType.DMA,
        ],
    )
    def deg_kernel(sd_hbm, out_hbm, idx_v, ones_v, zero_v, acc_sh, si0, si1):
        ci = lax.axis_index("c")
        si = lax.axis_index("s")
        sem = (si0, si1)

        @pl.loop(0, _K, step=16)
        def _(i):
            ones_v[pl.ds(i, 16)] = jnp.full((16,), 1.0, jnp.float32)

        @pl.when(si < 10)
        def _():
            @pl.loop(0, _RPS, step=16)
            def _(i):
                zero_v[pl.ds(i, 16)] = jnp.full((16,), 0.0, jnp.float32)

            pltpu.sync_copy(zero_v, acc_sh.at[pl.ds(si * _RPS, _RPS)])

        plsc.subcore_barrier()
        base = ci * _E + si * per_tile

        def idx_src(q):
            return sd_hbm.at[pl.ds(base + q * _K, _K)]

        pltpu.sync_copy(idx_src(0), idx_v.at[0])

        @pl.loop(0, n_grp)
        def _(grp):
            for b in (0, 1):
                nb = 1 - b
                q = 2 * grp + b
                if b == 0:
                    pltpu.async_copy(idx_src(q + 1), idx_v.at[nb], sem[nb])
                else:
                    @pl.when(grp < n_grp - 1)
                    def _():
                        pltpu.async_copy(idx_src(q + 1), idx_v.at[nb],
                                         sem[nb])

                if b == 0:
                    @pl.when(grp > 0)
                    def _():
                        pltpu.make_async_copy(idx_src(q), idx_v.at[b],
                                              sem[b]).wait()
                else:
                    pltpu.make_async_copy(idx_src(q), idx_v.at[b],
                                          sem[b]).wait()

                pltpu.sync_copy(ones_v, acc_sh.at[idx_v.at[b]], add=True)

        plsc.subcore_barrier()

        @pl.when(si < 10)
        def _():
            pltpu.sync_copy(acc_sh.at[pl.ds(si * _RPS, _RPS)], zero_v)
            pltpu.sync_copy(zero_v,
                            out_hbm.at[pl.ds(ci * _N + si * _RPS, _RPS)])

    return deg_kernel(sd)


def _sc_agg(g, src, dst, zeros_blk):
    per_tile = _E // (_NC * _NS)

    @functools.partial(
        pl.kernel,
        out_type=jax.ShapeDtypeStruct((2 * _N, _D), jnp.float32),
        mesh=plsc.VectorSubcoreMesh(**_MESH),
        scratch_types=[
            pltpu.VMEM((4, _K), jnp.int32),
            pltpu.VMEM((4, _K), jnp.int32),
            pltpu.VMEM((4, _K, _D), jnp.float32),
            pltpu.VMEM_SHARED((_N, _D), jnp.float32),
            [pltpu.SemaphoreType.DMA] * 4,
            [pltpu.SemaphoreType.DMA] * 4,
            [pltpu.SemaphoreType.DMA] * 4,
        ],
    )
    def agg_kernel(g_hbm, src_hbm, dst_hbm, zeros_hbm, out_hbm, sidx_v,
                   didx_v, rows_v, acc_sh, isem, gsem, ssem):
        ci = lax.axis_index("c")
        si = lax.axis_index("s")

        @pl.when(si < 10)
        def _():
            pltpu.sync_copy(zeros_hbm, acc_sh.at[pl.ds(si * _RPS, _RPS)])

        plsc.subcore_barrier()
        base = (ci * _NS + si) * per_tile

        def idx_fetch(q, s):
            off = base + q * _K
            pltpu.async_copy(src_hbm.at[pl.ds(off, _K)], sidx_v.at[s],
                             isem[s])
            pltpu.async_copy(dst_hbm.at[pl.ds(off, _K)], didx_v.at[s],
                             isem[s])

        def wait_idx(s):
            pltpu.make_async_copy(src_hbm.at[pl.ds(base, _K)],
                                  sidx_v.at[s], isem[s]).wait()
            pltpu.make_async_copy(dst_hbm.at[pl.ds(base, _K)],
                                  didx_v.at[s], isem[s]).wait()

        def gather(s):
            pltpu.async_copy(g_hbm.at[sidx_v.at[s]], rows_v.at[s], gsem[s])

        def wait_gather(s):
            pltpu.make_async_copy(g_hbm.at[sidx_v.at[s]], rows_v.at[s],
                                  gsem[s]).wait()

        def scatter(s):
            pltpu.async_copy(rows_v.at[s], acc_sh.at[didx_v.at[s]],
                             ssem[s], add=True)

        def wait_scatter(s):
            pltpu.make_async_copy(rows_v.at[s], acc_sh.at[didx_v.at[s]],
                                  ssem[s]).wait()

        idx_fetch(0, 0)
        idx_fetch(1, 1)
        wait_idx(0)
        gather(0)

        @pl.loop(0, (_CPT - 1) // 4)
        def _(grp):
            for b in range(4):
                s0 = b
                s1 = (b + 1) % 4
                s2 = (b + 2) % 4

                if b < 2:
                    @pl.when(grp > 0)
                    def _():
                        wait_scatter(s2)
                else:
                    wait_scatter(s2)

                if b == 3:
                    @pl.when(grp < (_CPT - 1) // 4 - 1)
                    def _():
                        idx_fetch(4 * grp + b + 2, s2)
                else:
                    idx_fetch(4 * grp + b + 2, s2)

                wait_gather(s0)
                scatter(s0)
                wait_idx(s1)
                gather(s1)

        wait_scatter(2)
        wait_gather(0)
        scatter(0)
        wait_scatter(3)
        wait_scatter(0)
        plsc.subcore_barrier()

        @pl.when(si < 10)
        def _():
            pltpu.sync_copy(acc_sh.at[pl.ds(si * _RPS, _RPS)],
                            out_hbm.at[pl.ds(ci * _N + si * _RPS, _RPS)])

    return agg_kernel(g, src, dst, zeros_blk)


def _norm(deg):
    return jnp.where(deg > 0, lax.rsqrt(jnp.maximum(deg, 1.0)), 0.0)


def _mm(a, b):
    return lax.dot_general(a, b, (((1,), (0,)), ((), ())),
                           precision=lax.Precision.HIGHEST,
                           preferred_element_type=jnp.float32)


def _tc_mm_scale(x, W, degout):
    def body(x_ref, w_ref, d_ref, o_ref):
        o_ref[...] = _mm(x_ref[...], w_ref[...]) * _norm(d_ref[...])

    return pl.pallas_call(
        body,
        grid=(_N // _RB,),
        in_specs=[pl.BlockSpec((_RB, _D), lambda i: (i, 0)),
                  pl.BlockSpec((_D, _D), lambda i: (0, 0)),
                  pl.BlockSpec((_RB, 1), lambda i: (i, 0))],
        out_specs=pl.BlockSpec((_RB, _D), lambda i: (i, 0)),
        out_shape=jax.ShapeDtypeStruct((_N, _D), jnp.float32),
    )(x, W, degout)


def _tc_mid(agg, degin, b1, W2, degout):
    def body(a_ref, di_ref, b_ref, w_ref, do_ref, o_ref):
        s = a_ref[0] + a_ref[1]
        h = jnp.maximum(s * _norm(di_ref[...]) + b_ref[...], 0.0)
        o_ref[...] = _mm(h, w_ref[...]) * _norm(do_ref[...])

    return pl.pallas_call(
        body,
        grid=(_N // _RB,),
        in_specs=[pl.BlockSpec((_NC, _RB, _D), lambda i: (0, i, 0)),
                  pl.BlockSpec((_RB, 1), lambda i: (i, 0)),
                  pl.BlockSpec((1, _D), lambda i: (0, 0)),
                  pl.BlockSpec((_D, _D), lambda i: (0, 0)),
                  pl.BlockSpec((_RB, 1), lambda i: (i, 0))],
        out_specs=pl.BlockSpec((_RB, _D), lambda i: (i, 0)),
        out_shape=jax.ShapeDtypeStruct((_N, _D), jnp.float32),
    )(agg, degin, b1, W2, degout)


def _tc_fin(agg, degin, b2):
    def body(a_ref, di_ref, b_ref, o_ref):
        o_ref[...] = (a_ref[0] + a_ref[1]) * _norm(di_ref[...]) + b_ref[...]

    return pl.pallas_call(
        body,
        grid=(_N // _RB,),
        in_specs=[pl.BlockSpec((_NC, _RB, _D), lambda i: (0, i, 0)),
                  pl.BlockSpec((_RB, 1), lambda i: (i, 0)),
                  pl.BlockSpec((1, _D), lambda i: (0, 0))],
        out_specs=pl.BlockSpec((_RB, _D), lambda i: (i, 0)),
        out_shape=jax.ShapeDtypeStruct((_N, _D), jnp.float32),
    )(agg, degin, b2)


def kernel(x, edge_index, W1, b1, W2, b2):
    src = edge_index[0]
    dst = edge_index[1]
    sd = jnp.concatenate([src, dst])
    zeros_blk = jnp.zeros((_RPS, _D), jnp.float32)

    deg = _sc_degrees(sd).reshape(_NC, _N, 1)
    degout = deg[0]
    degin = deg[1]

    g1 = _tc_mm_scale(x, W1, degout)
    agg1 = _sc_agg(g1, src, dst, zeros_blk).reshape(_NC, _N, _D)
    g2 = _tc_mid(agg1, degin, b1.reshape(1, _D), W2, degout)
    agg2 = _sc_agg(g2, src, dst, zeros_blk).reshape(_NC, _N, _D)
    return _tc_fin(agg2, degin, b2.reshape(1, _D))


# speedup vs baseline: 2.7687x; 1.0119x over previous
import functools

import jax
import jax.numpy as jnp
from jax import lax
from jax.experimental import pallas as pl
from jax.experimental.pallas import tpu as pltpu
from jax.experimental.pallas import tpu_sc as plsc

_N = 10000
_E = 320000
_D = 128
_NC = 2
_NS = 16
_K = 80
_CPT = _E // (_NC * _NS * _K)
_RPS = 1000
_RB = 1000

_MESH = dict(core_axis_name="c", subcore_axis_name="s", num_cores=_NC,
             num_subcores=_NS)


def _sc_degrees(sd):
    per_tile = _E // _NS
    n_chunks = per_tile // _K
    n_grp = n_chunks // 2

    @functools.partial(
        pl.kernel,
        out_type=jax.ShapeDtypeStruct((2 * _N,), jnp.float32),
        mesh=plsc.VectorSubcoreMesh(**_MESH),
        scratch_types=[
            pltpu.VMEM((2, _K), jnp.int32),
            pltpu.VMEM((_K,), jnp.float32),
            pltpu.VMEM((_RPS,), jnp.float32),
            pltpu.VMEM_SHARED((_N,), jnp.float32),
            pltpu.SemaphoreType.DMA,
            pltpu.SemaphoreType.DMA,
        ],
    )
    def deg_kernel(sd_hbm, out_hbm, idx_v, ones_v, zero_v, acc_sh, si0, si1):
        ci = lax.axis_index("c")
        si = lax.axis_index("s")
        sem = (si0, si1)

        @pl.loop(0, _K, step=16)
        def _(i):
            ones_v[pl.ds(i, 16)] = jnp.full((16,), 1.0, jnp.float32)

        @pl.when(si < 10)
        def _():
            @pl.loop(0, _RPS, step=16)
            def _(i):
                zero_v[pl.ds(i, 16)] = jnp.full((16,), 0.0, jnp.float32)

            pltpu.sync_copy(zero_v, acc_sh.at[pl.ds(si * _RPS, _RPS)])

        plsc.subcore_barrier()
        base = ci * _E + si * per_tile

        def idx_src(q):
            return sd_hbm.at[pl.ds(base + q * _K, _K)]

        pltpu.sync_copy(idx_src(0), idx_v.at[0])

        @pl.loop(0, n_grp)
        def _(grp):
            for b in (0, 1):
                nb = 1 - b
                q = 2 * grp + b
                if b == 0:
                    pltpu.async_copy(idx_src(q + 1), idx_v.at[nb], sem[nb])
                else:
                    @pl.when(grp < n_grp - 1)
                    def _():
                        pltpu.async_copy(idx_src(q + 1), idx_v.at[nb],
                                         sem[nb])

                if b == 0:
                    @pl.when(grp > 0)
                    def _():
                        pltpu.make_async_copy(idx_src(q), idx_v.at[b],
                                              sem[b]).wait()
                else:
                    pltpu.make_async_copy(idx_src(q), idx_v.at[b],
                                          sem[b]).wait()

                pltpu.sync_copy(ones_v, acc_sh.at[idx_v.at[b]], add=True)

        plsc.subcore_barrier()

        @pl.when(si < 10)
        def _():
            pltpu.sync_copy(acc_sh.at[pl.ds(si * _RPS, _RPS)], zero_v)
            pltpu.sync_copy(zero_v,
                            out_hbm.at[pl.ds(ci * _N + si * _RPS, _RPS)])

    return deg_kernel(sd)


def _sc_agg(g, src, dst, zeros_blk):
    per_tile = _E // (_NC * _NS)

    @functools.partial(
        pl.kernel,
        out_type=jax.ShapeDtypeStruct((2 * _N, _D), jnp.float32),
        mesh=plsc.VectorSubcoreMesh(**_MESH),
        scratch_types=[
            pltpu.VMEM((4, _K), jnp.int32),
            pltpu.VMEM((4, _K), jnp.int32),
            pltpu.VMEM((4, _K, _D), jnp.float32),
            pltpu.VMEM_SHARED((_N, _D), jnp.float32),
            [pltpu.SemaphoreType.DMA] * 4,
            [pltpu.SemaphoreType.DMA] * 4,
            [pltpu.SemaphoreType.DMA] * 4,
        ],
    )
    def agg_kernel(g_hbm, src_hbm, dst_hbm, zeros_hbm, out_hbm, sidx_v,
                   didx_v, rows_v, acc_sh, isem, gsem, ssem):
        ci = lax.axis_index("c")
        si = lax.axis_index("s")

        @pl.when(si < 10)
        def _():
            pltpu.sync_copy(zeros_hbm, acc_sh.at[pl.ds(si * _RPS, _RPS)])

        plsc.subcore_barrier()
        base = (ci * _NS + si) * per_tile

        def idx_fetch(q, s):
            off = base + q * _K
            pltpu.async_copy(src_hbm.at[pl.ds(off, _K)], sidx_v.at[s],
                             isem[s])
            pltpu.async_copy(dst_hbm.at[pl.ds(off, _K)], didx_v.at[s],
                             isem[s])

        def wait_idx(s):
            pltpu.make_async_copy(src_hbm.at[pl.ds(base, _K)],
                                  sidx_v.at[s], isem[s]).wait()
            pltpu.make_async_copy(dst_hbm.at[pl.ds(base, _K)],
                                  didx_v.at[s], isem[s]).wait()

        def gather(s):
            pltpu.async_copy(g_hbm.at[sidx_v.at[s]], rows_v.at[s], gsem[s])

        def wait_gather(s):
            pltpu.make_async_copy(g_hbm.at[sidx_v.at[s]], rows_v.at[s],
                                  gsem[s]).wait()

        def scatter(s):
            pltpu.async_copy(rows_v.at[s], acc_sh.at[didx_v.at[s]],
                             ssem[s], add=True)

        def wait_scatter(s):
            pltpu.make_async_copy(rows_v.at[s], acc_sh.at[didx_v.at[s]],
                                  ssem[s]).wait()

        idx_fetch(0, 0)
        idx_fetch(1, 1)
        wait_idx(0)
        gather(0)

        @pl.loop(0, (_CPT - 1) // 4)
        def _(grp):
            for b in range(4):
                s0 = b
                s1 = (b + 1) % 4
                s2 = (b + 2) % 4

                if b < 2:
                    @pl.when(grp > 0)
                    def _():
                        wait_scatter(s2)
                else:
                    wait_scatter(s2)

                if b == 3:
                    @pl.when(grp < (_CPT - 1) // 4 - 1)
                    def _():
                        idx_fetch(4 * grp + b + 2, s2)
                else:
                    idx_fetch(4 * grp + b + 2, s2)

                wait_gather(s0)
                scatter(s0)
                wait_idx(s1)
                gather(s1)

        wait_scatter(2)
        wait_gather(0)
        scatter(0)
        wait_scatter(3)
        wait_scatter(0)
        plsc.subcore_barrier()

        @pl.when(si < 10)
        def _():
            pltpu.sync_copy(acc_sh.at[pl.ds(si * _RPS, _RPS)],
                            out_hbm.at[pl.ds(ci * _N + si * _RPS, _RPS)])

    return agg_kernel(g, src, dst, zeros_blk)


def _norm(deg):
    return jnp.where(deg > 0, lax.rsqrt(jnp.maximum(deg, 1.0)), 0.0)


def _mm(a, b):
    return lax.dot_general(a, b, (((1,), (0,)), ((), ())),
                           precision=lax.Precision.HIGHEST,
                           preferred_element_type=jnp.float32)


def _tc_mm(x, W):
    def body(x_ref, w_ref, o_ref):
        o_ref[...] = _mm(x_ref[...], w_ref[...])

    return pl.pallas_call(
        body,
        grid=(_N // _RB,),
        in_specs=[pl.BlockSpec((_RB, _D), lambda i: (i, 0)),
                  pl.BlockSpec((_D, _D), lambda i: (0, 0))],
        out_specs=pl.BlockSpec((_RB, _D), lambda i: (i, 0)),
        out_shape=jax.ShapeDtypeStruct((_N, _D), jnp.float32),
    )(x, W)


def _tc_scale(p, degout):
    def body(p_ref, d_ref, o_ref):
        o_ref[...] = p_ref[...] * _norm(d_ref[...])

    return pl.pallas_call(
        body,
        grid=(_N // _RB,),
        in_specs=[pl.BlockSpec((_RB, _D), lambda i: (i, 0)),
                  pl.BlockSpec((_RB, 1), lambda i: (i, 0))],
        out_specs=pl.BlockSpec((_RB, _D), lambda i: (i, 0)),
        out_shape=jax.ShapeDtypeStruct((_N, _D), jnp.float32),
    )(p, degout)


def _tc_mid(agg, degin, b1, W2, degout):
    def body(a_ref, di_ref, b_ref, w_ref, do_ref, o_ref):
        s = a_ref[0] + a_ref[1]
        h = jnp.maximum(s * _norm(di_ref[...]) + b_ref[...], 0.0)
        o_ref[...] = _mm(h, w_ref[...]) * _norm(do_ref[...])

    return pl.pallas_call(
        body,
        grid=(_N // _RB,),
        in_specs=[pl.BlockSpec((_NC, _RB, _D), lambda i: (0, i, 0)),
                  pl.BlockSpec((_RB, 1), lambda i: (i, 0)),
                  pl.BlockSpec((1, _D), lambda i: (0, 0)),
                  pl.BlockSpec((_D, _D), lambda i: (0, 0)),
                  pl.BlockSpec((_RB, 1), lambda i: (i, 0))],
        out_specs=pl.BlockSpec((_RB, _D), lambda i: (i, 0)),
        out_shape=jax.ShapeDtypeStruct((_N, _D), jnp.float32),
    )(agg, degin, b1, W2, degout)


def _tc_fin(agg, degin, b2):
    def body(a_ref, di_ref, b_ref, o_ref):
        o_ref[...] = (a_ref[0] + a_ref[1]) * _norm(di_ref[...]) + b_ref[...]

    return pl.pallas_call(
        body,
        grid=(_N // _RB,),
        in_specs=[pl.BlockSpec((_NC, _RB, _D), lambda i: (0, i, 0)),
                  pl.BlockSpec((_RB, 1), lambda i: (i, 0)),
                  pl.BlockSpec((1, _D), lambda i: (0, 0))],
        out_specs=pl.BlockSpec((_RB, _D), lambda i: (i, 0)),
        out_shape=jax.ShapeDtypeStruct((_N, _D), jnp.float32),
    )(agg, degin, b2)


def kernel(x, edge_index, W1, b1, W2, b2):
    src = edge_index[0]
    dst = edge_index[1]
    sd = jnp.concatenate([src, dst])
    zeros_blk = jnp.zeros((_RPS, _D), jnp.float32)

    deg = _sc_degrees(sd).reshape(_NC, _N, 1)
    degout = deg[0]
    degin = deg[1]

    p1 = _tc_mm(x, W1)
    g1 = _tc_scale(p1, degout)
    agg1 = _sc_agg(g1, src, dst, zeros_blk).reshape(_NC, _N, _D)
    g2 = _tc_mid(agg1, degin, b1.reshape(1, _D), W2, degout)
    agg2 = _sc_agg(g2, src, dst, zeros_blk).reshape(_NC, _N, _D)
    return _tc_fin(agg2, degin, b2.reshape(1, _D))
